# baseline (speedup 1.0000x reference)
import functools

import jax
import jax.numpy as jnp
from jax import lax
from jax.experimental import pallas as pl
from jax.experimental.pallas import tpu as pltpu
from jax.experimental.pallas import tpu_sc as plsc

D_MODEL = 64
NC = 2
NS = 16
NW = NC * NS
BB = 128
NDT = D_MODEL // 8


def _positional_encoding(seq_len, d_model):
    pos = jnp.arange(0, seq_len, dtype=jnp.float32)[:, None]
    dim = jnp.arange(0, d_model, dtype=jnp.float32)
    result = jnp.zeros((seq_len, d_model), dtype=jnp.float32)
    sin_part = jnp.sin(pos / (10000.0 ** (dim[0::2] / d_model)))
    cos_part = jnp.cos(pos / (10000.0 ** (dim[1::2] / d_model)))
    result = result.at[:, 0::2].set(sin_part)
    result = result.at[:, 1::2].set(cos_part)
    return result


@jax.jit
def _prep(tt, tail_pad):
    mesh = plsc.VectorSubcoreMesh(
        core_axis_name="c", subcore_axis_name="s", num_cores=NC,
        num_subcores=NS)
    nv = tt.shape[1]
    n_units = nv // BB
    tail = nv - n_units * BB

    @functools.partial(
        pl.kernel,
        out_type=jax.ShapeDtypeStruct((nv, BB), jnp.float32),
        mesh=mesh,
        scratch_types=[
            pltpu.VMEM((D_MODEL, BB), jnp.float32),
            pltpu.VMEM((D_MODEL, BB), jnp.float32),
            pltpu.VMEM((BB, BB + 1), jnp.float32),
            pltpu.VMEM((BB, BB + 1), jnp.float32),
            pltpu.SemaphoreType.DMA,
            pltpu.SemaphoreType.DMA,
            pltpu.SemaphoreType.DMA,
            pltpu.SemaphoreType.DMA,
        ],
        compiler_params=pltpu.CompilerParams(
            use_tc_tiling_on_sc=True, needs_layout_passes=False),
    )
    def k(tt_hbm, tail_hbm, out_hbm, db0, db1, vb0, vb1, gsem0, gsem1,
          ssem0, ssem1):
        cid = lax.axis_index("c")
        sid = lax.axis_index("s")
        wid = sid * NC + cid
        db = (db0, db1)
        vb = (vb0, vb1)
        gsem = (gsem0, gsem1)
        ssem = (ssem0, ssem1)
        lanes = lax.iota(jnp.int32, 16)
        rows_g = [(lanes + 16 * g) for g in range(8)]

        per_w = (n_units + NW - 1) // NW

        def fire(u, p):
            pltpu.async_copy(tt_hbm.at[:, pl.ds(u * BB, BB)], db[p], gsem[p])

        def drain_gather(p):
            pltpu.make_async_copy(tt_hbm.at[:, pl.ds(0, BB)], db[p],
                                  gsem[p]).wait()

        def transform(p):
            dbp, vbp = db[p], vb[p]

            @plsc.parallel_loop(0, D_MODEL, unroll=4)
            def _(d):
                colv = (lanes & 0) + d
                for g in range(8):
                    v = dbp[d, pl.ds(16 * g, 16)]
                    plsc.store_scatter(vbp, [rows_g[g], colv], v)

        def store(u, p):
            pltpu.async_copy(vb[p].at[:, pl.ds(0, BB)],
                             out_hbm.at[pl.ds(u * BB, BB)], ssem[p])

        def drain_store(p):
            pltpu.make_async_copy(vb[p].at[:, pl.ds(0, BB)],
                                  out_hbm.at[pl.ds(0, BB)], ssem[p]).wait()

        u0 = wid * per_w
        n_mine = jnp.minimum(per_w, jnp.maximum(n_units - u0, 0))

        @pl.when(n_mine > 0)
        def _():
            fire(u0, 0)

            def pair(kk, _):
                i0 = 2 * kk
                i1 = i0 + 1

                @pl.when(i1 < n_mine)
                def _():
                    fire(u0 + i1, 1)

                @pl.when(i0 < n_mine)
                def _():
                    drain_gather(0)

                    @pl.when(kk > 0)
                    def _():
                        drain_store(0)

                    transform(0)
                    store(u0 + i0, 0)

                @pl.when(i0 + 2 < n_mine)
                def _():
                    fire(u0 + i0 + 2, 0)

                @pl.when(i1 < n_mine)
                def _():
                    drain_gather(1)

                    @pl.when(kk > 0)
                    def _():
                        drain_store(1)

                    transform(1)
                    store(u0 + i1, 1)
                return ()

            lax.fori_loop(0, (per_w + 1) // 2, pair, ())

            @pl.when(n_mine > 0)
            def _():
                drain_store(0)

            @pl.when(n_mine > 1)
            def _():
                drain_store(1)

        if tail:
            @pl.when(wid == 0)
            def _():
                pltpu.sync_copy(tail_hbm, db[0])
                pltpu.sync_copy(db[0].at[pl.ds(0, tail), :],
                                out_hbm.at[pl.ds(n_units * BB, tail)])

    return k(tt, tail_pad)


@functools.partial(jax.jit, static_argnames=("seq_len", "n_batch"))
def _run(xt, pe, table, seq_len, n_batch):
    mesh = plsc.VectorSubcoreMesh(
        core_axis_name="c", subcore_axis_name="s", num_cores=NC,
        num_subcores=NS)
    nbt = n_batch // BB
    n_pairs = seq_len // 2

    @functools.partial(
        pl.kernel,
        out_type=jax.ShapeDtypeStruct((seq_len, NDT, nbt, 8, BB),
                                      jnp.float32),
        mesh=mesh,
        scratch_types=[
            pltpu.VMEM((seq_len, BB), jnp.int32),
            pltpu.VMEM((seq_len, D_MODEL), jnp.float32),
            pltpu.VMEM((BB, 2 * D_MODEL), jnp.float32),
            pltpu.VMEM((BB, 2 * D_MODEL), jnp.float32),
            pltpu.VMEM((NDT, 8, BB + 1), jnp.float32),
            pltpu.VMEM((NDT, 8, BB + 1), jnp.float32),
            pltpu.SemaphoreType.DMA,
            pltpu.SemaphoreType.DMA,
            pltpu.SemaphoreType.DMA,
            pltpu.SemaphoreType.DMA,
        ],
        compiler_params=pltpu.CompilerParams(
            use_tc_tiling_on_sc=False, needs_layout_passes=False),
    )
    def k(xt_hbm, pe_hbm, table_hbm, out_hbm, idx_v, pe_v, rb0, rb1, tb0,
          tb1, gsem0, gsem1, ssem0, ssem1):
        cid = lax.axis_index("c")
        sid = lax.axis_index("s")
        wid = sid * NC + cid
        rb = (rb0, rb1)
        tb = (tb0, tb1)
        gsem = (gsem0, gsem1)
        ssem = (ssem0, ssem1)

        pltpu.sync_copy(xt_hbm.at[:, pl.ds(wid * BB, BB)], idx_v)
        pltpu.sync_copy(pe_hbm, pe_v)

        lanes = lax.iota(jnp.int32, 16)
        dt_idx = [(lanes + 16 * j) >> 3 for j in range(4)]
        di_idx = [(lanes + 16 * j) & 7 for j in range(4)]

        def fire(s, p):
            pltpu.async_copy(table_hbm.at[idx_v.at[s]], rb[p], gsem[p])

        def drain_gather(p):
            pltpu.make_async_copy(table_hbm.at[idx_v.at[0]], rb[p],
                                  gsem[p]).wait()

        def transform(s, p):
            pe_j = [pe_v[s, pl.ds(16 * j, 16)] for j in range(4)]
            rbp, tbp = rb[p], tb[p]

            @plsc.parallel_loop(0, BB, unroll=8)
            def _(b):
                colv = (lanes & 0) + b
                for j in range(4):
                    v = rbp[b, pl.ds(16 * j, 16)] + pe_j[j]
                    plsc.store_scatter(tbp, [dt_idx[j], di_idx[j], colv], v)

        def store(s, p):
            pltpu.async_copy(tb[p].at[:, :, pl.ds(0, BB)],
                             out_hbm.at[s, :, wid], ssem[p])

        def drain_store(p):
            pltpu.make_async_copy(tb[p].at[:, :, pl.ds(0, BB)],
                                  out_hbm.at[0, :, wid], ssem[p]).wait()

        fire(0, 0)

        def pair_body(kk, _):
            s0 = 2 * kk
            s1 = s0 + 1
            fire(s1, 1)
            drain_gather(0)

            @pl.when(kk > 0)
            def _():
                drain_store(0)

            transform(s0, 0)
            store(s0, 0)

            @pl.when(kk < n_pairs - 1)
            def _():
                fire(s0 + 2, 0)

            drain_gather(1)

            @pl.when(kk > 0)
            def _():
                drain_store(1)

            transform(s1, 1)
            store(s1, 1)
            return ()

        lax.fori_loop(0, n_pairs, pair_body, ())
        drain_store(0)
        drain_store(1)

    return k(xt, pe, table)


def kernel(x, table):
    b, s = x.shape
    xt = jnp.transpose(x).astype(jnp.int32)
    pe = _positional_encoding(s, D_MODEL)
    n_tail = table.shape[0] % BB
    tail_pad = jnp.pad(table[table.shape[0] - n_tail:],
                       ((0, D_MODEL - n_tail), (0, BB - D_MODEL)))
    table_p = _prep(jnp.transpose(table), tail_pad)
    out5 = _run(xt, pe, table_p, s, b)
    return out5.transpose((2, 4, 0, 1, 3)).reshape(b, s, D_MODEL)


# speedup vs baseline: 1.4178x; 1.4178x over previous
import functools

import jax
import jax.numpy as jnp
from jax import lax
from jax.experimental import pallas as pl
from jax.experimental.pallas import tpu as pltpu
from jax.experimental.pallas import tpu_sc as plsc

D_MODEL = 64
NC = 2
NS = 16
NW = NC * NS
BB = 128
NDT = D_MODEL // 8


def _positional_encoding(seq_len, d_model):
    pos = jnp.arange(0, seq_len, dtype=jnp.float32)[:, None]
    dim = jnp.arange(0, d_model, dtype=jnp.float32)
    result = jnp.zeros((seq_len, d_model), dtype=jnp.float32)
    sin_part = jnp.sin(pos / (10000.0 ** (dim[0::2] / d_model)))
    cos_part = jnp.cos(pos / (10000.0 ** (dim[1::2] / d_model)))
    result = result.at[:, 0::2].set(sin_part)
    result = result.at[:, 1::2].set(cos_part)
    return result


@functools.partial(jax.jit, static_argnames=("seq_len", "n_batch"))
def _run(xt, pe, table, seq_len, n_batch):
    mesh = plsc.VectorSubcoreMesh(
        core_axis_name="c", subcore_axis_name="s", num_cores=NC,
        num_subcores=NS)
    nbt = n_batch // BB
    n_pairs = seq_len // 2

    @functools.partial(
        pl.kernel,
        out_type=jax.ShapeDtypeStruct((seq_len, NDT, nbt, 8, BB),
                                      jnp.float32),
        mesh=mesh,
        scratch_types=[
            pltpu.VMEM((seq_len, BB), jnp.int32),
            pltpu.VMEM((seq_len, D_MODEL), jnp.float32),
            pltpu.VMEM((BB, 2 * D_MODEL), jnp.float32),
            pltpu.VMEM((BB, 2 * D_MODEL), jnp.float32),
            pltpu.VMEM((NDT, 8, BB + 1), jnp.float32),
            pltpu.VMEM((NDT, 8, BB + 1), jnp.float32),
            pltpu.SemaphoreType.DMA,
            pltpu.SemaphoreType.DMA,
            pltpu.SemaphoreType.DMA,
            pltpu.SemaphoreType.DMA,
        ],
        compiler_params=pltpu.CompilerParams(
            use_tc_tiling_on_sc=False, needs_layout_passes=False),
    )
    def k(xt_hbm, pe_hbm, table_hbm, out_hbm, idx_v, pe_v, rb0, rb1, tb0,
          tb1, gsem0, gsem1, ssem0, ssem1):
        cid = lax.axis_index("c")
        sid = lax.axis_index("s")
        wid = sid * NC + cid
        rb = (rb0, rb1)
        tb = (tb0, tb1)
        gsem = (gsem0, gsem1)
        ssem = (ssem0, ssem1)

        pltpu.sync_copy(xt_hbm.at[:, pl.ds(wid * BB, BB)], idx_v)
        pltpu.sync_copy(pe_hbm, pe_v)

        lanes = lax.iota(jnp.int32, 16)
        dt_idx = [(lanes + 16 * j) >> 3 for j in range(4)]
        di_idx = [(lanes + 16 * j) & 7 for j in range(4)]

        def fire(s, p):
            pltpu.async_copy(table_hbm.at[idx_v.at[s]], rb[p], gsem[p])

        def drain_gather(p):
            pltpu.make_async_copy(table_hbm.at[idx_v.at[0]], rb[p],
                                  gsem[p]).wait()

        def transform(s, p):
            pe_j = [pe_v[s, pl.ds(16 * j, 16)] for j in range(4)]
            rbp, tbp = rb[p], tb[p]

            @plsc.parallel_loop(0, BB, unroll=8)
            def _(b):
                colv = (lanes & 0) + b
                for j in range(4):
                    v = rbp[b, pl.ds(16 * j, 16)] + pe_j[j]
                    plsc.store_scatter(tbp, [dt_idx[j], di_idx[j], colv], v)

        def store(s, p):
            pltpu.async_copy(tb[p].at[:, :, pl.ds(0, BB)],
                             out_hbm.at[s, :, wid], ssem[p])

        def drain_store(p):
            pltpu.make_async_copy(tb[p].at[:, :, pl.ds(0, BB)],
                                  out_hbm.at[0, :, wid], ssem[p]).wait()

        fire(0, 0)

        def pair_body(kk, _):
            s0 = 2 * kk
            s1 = s0 + 1
            fire(s1, 1)
            drain_gather(0)

            @pl.when(kk > 0)
            def _():
                drain_store(0)

            transform(s0, 0)
            store(s0, 0)

            @pl.when(kk < n_pairs - 1)
            def _():
                fire(s0 + 2, 0)

            drain_gather(1)

            @pl.when(kk > 0)
            def _():
                drain_store(1)

            transform(s1, 1)
            store(s1, 1)
            return ()

        lax.fori_loop(0, n_pairs, pair_body, ())
        drain_store(0)
        drain_store(1)

    return k(xt, pe, table)


def kernel(x, table):
    b, s = x.shape
    xt = jnp.transpose(x).astype(jnp.int32)
    pe = _positional_encoding(s, D_MODEL)
    table_p = jnp.pad(table, ((0, 0), (0, 2 * D_MODEL - table.shape[1])))
    out5 = _run(xt, pe, table_p, s, b)
    return out5.transpose((2, 4, 0, 1, 3)).reshape(b, s, D_MODEL)
